# baseline (speedup 1.0000x reference)
import functools

import jax
import jax.numpy as jnp
from jax import lax
from jax.experimental import pallas as pl
import jax.experimental.pallas.tpu as pltpu
from jax.experimental.pallas import tpu_sc as plsc

EPS = 1e-5

NC = 2
NS = 16
NW = NC * NS
CH = 128
CAP = 4096
EPAD = 32
GK = 16


def _prepass_body(adj_ref, sel_ref, dis_ref, cnt_ref, *, nsteps):
    i = pl.program_id(0)
    blk = adj_ref[...]
    part = jnp.sum(blk, axis=0, keepdims=True)

    @pl.when(i == 0)
    def _():
        dis_ref[...] = part

    @pl.when(i > 0)
    def _():
        dis_ref[...] += part

    @pl.when(i == nsteps - 1)
    def _():
        dis_ref[...] = lax.rsqrt(1.0 + dis_ref[...])

    nz = (blk != 0.0).astype(jnp.bfloat16)
    cnt = jnp.dot(nz, sel_ref[...], preferred_element_type=jnp.float32)
    cnt_ref[...] = cnt.astype(jnp.int32)


def _prepass(adj, sel, rb):
    n = adj.shape[0]
    nsteps = n // rb
    nch = n // CH
    return pl.pallas_call(
        functools.partial(_prepass_body, nsteps=nsteps),
        grid=(nsteps,),
        in_specs=[
            pl.BlockSpec((rb, n), lambda i: (i, 0)),
            pl.BlockSpec((n, nch), lambda i: (0, 0)),
        ],
        out_specs=[
            pl.BlockSpec((1, n), lambda i: (0, 0)),
            pl.BlockSpec((rb, nch), lambda i: (i, 0)),
        ],
        out_shape=[
            jax.ShapeDtypeStruct((1, n), jnp.float32),
            jax.ShapeDtypeStruct((n, nch), jnp.int32),
        ],
    )(adj, sel)


def _xpose_body(x_ref, o_ref):
    o_ref[...] = x_ref[...]


def _xpose(x, nb):
    b, n, f = x.shape
    nblk = n // nb
    return pl.pallas_call(
        _xpose_body,
        grid=(b, nblk),
        in_specs=[pl.BlockSpec((nb, f), lambda bi, ni: (bi * nblk + ni, 0))],
        out_specs=pl.BlockSpec((nb, f), lambda bi, ni: (ni, bi)),
        out_shape=jax.ShapeDtypeStruct((n, b * f), jnp.float32),
    )(x.reshape(b * n, f))


def _cstore(ref, base, x, msk):
    cs = plsc.cumsum(msk.astype(jnp.int32))
    plsc.store_scatter(ref, [base + cs - 1], x, mask=msk)
    return cs[15]


def _extract_body(cnt_hbm, adj1m_hbm, eidx_hbm, evals_hbm, ecnt_hbm,
                  cntbuf, cidA, cidB, chk, ibuf, vbuf, c16, *, n, rpw):
    cidx = lax.axis_index("c")
    sidx = lax.axis_index("s")
    wid = sidx * NC + cidx
    row0 = wid * rpw
    ncpr = n // CH
    halfn = n // 2
    iota = lax.iota(jnp.int32, 16)
    zeros_i = jnp.zeros((16,), jnp.int32)
    zeros_f = jnp.zeros((16,), jnp.float32)

    pltpu.sync_copy(cnt_hbm.at[pl.ds(row0, rpw)], cntbuf)

    def _cchunk(r, carry):
        cA, cB = carry
        for k in range(ncpr // 16):
            v = cntbuf[r, pl.ds(k * 16, 16)]
            msk = v > 0
            ids = (row0 + r) * ncpr + k * 16 + iota
            if k < ncpr // 32:
                cA = cA + _cstore(cidA, cA, ids, msk)
            else:
                cB = cB + _cstore(cidB, cB, ids, msk)
        return (cA, cB)

    cA, cB = lax.fori_loop(0, rpw, _cchunk,
                           (jnp.int32(0), jnp.int32(0)))

    for half, (cid, cptr) in enumerate(((cidA, cA), (cidB, cB))):
        cpad = (cptr + GK - 1) // GK * GK
        _cstore(cid, cptr, zeros_i, iota < (cpad - cptr))
        ngr = cpad // GK

        def _group(g, eptr):
            pltpu.sync_copy(adj1m_hbm.at[cid.at[pl.ds(g * GK, GK)]], chk)
            cvec = cid[pl.ds(g * GK, GK)]
            for j in range(GK):
                ce = g * GK + j
                live = ce < cptr
                cj = cvec[j]
                base = cj * CH - row0 * n
                for jj in range(CH // 16):
                    x = chk[j, pl.ds(jj * 16, 16)]
                    msk = (x != 0.0) & live & (eptr < CAP)
                    _cstore(vbuf, eptr, x, msk)
                    ids = base + jj * 16 + iota
                    eptr = eptr + _cstore(ibuf, eptr, ids, msk)
            return eptr

        eptr = lax.fori_loop(0, ngr, _group, jnp.int32(0))

        pad_id = zeros_i + half * halfn
        epad = (eptr + EPAD - 1) // EPAD * EPAD
        padn = epad - eptr
        m1 = iota < jnp.minimum(padn, 16)
        _cstore(ibuf, eptr, pad_id, m1)
        _cstore(vbuf, eptr, zeros_f, m1)
        e2 = eptr + jnp.minimum(padn, 16)
        m2 = iota < (padn - 16)
        _cstore(ibuf, e2, pad_id, m2)
        _cstore(vbuf, e2, zeros_f, m2)

        pltpu.sync_copy(ibuf.at[pl.ds(0, CAP)], eidx_hbm.at[wid, half])
        pltpu.sync_copy(vbuf.at[pl.ds(0, CAP)], evals_hbm.at[wid, half])
        lane = jnp.where(iota == half, epad, 0).astype(jnp.int32)
        if half == 0:
            c16[...] = lane
        else:
            c16[...] += lane
    pltpu.sync_copy(c16, ecnt_hbm.at[wid])


def _extract(cnt64, adj1m, n):
    rpw = n // NW
    nhchunk = rpw * (n // CH) // 2
    mesh = plsc.VectorSubcoreMesh(core_axis_name="c", subcore_axis_name="s",
                                  num_cores=NC, num_subcores=NS)
    kfn = pl.kernel(
        functools.partial(_extract_body, n=n, rpw=rpw),
        out_type=[
            jax.ShapeDtypeStruct((NW, 2, CAP), jnp.int32),
            jax.ShapeDtypeStruct((NW, 2, CAP), jnp.float32),
            jax.ShapeDtypeStruct((NW, 16), jnp.int32),
        ],
        mesh=mesh,
        compiler_params=pltpu.CompilerParams(needs_layout_passes=False),
        scratch_types=[
            pltpu.VMEM((rpw, n // CH), jnp.int32),
            pltpu.VMEM((nhchunk + 16,), jnp.int32),
            pltpu.VMEM((nhchunk + 16,), jnp.int32),
            pltpu.VMEM((GK, CH), jnp.float32),
            pltpu.VMEM((CAP + 16,), jnp.int32),
            pltpu.VMEM((CAP + 16,), jnp.float32),
            pltpu.VMEM((16,), jnp.int32),
        ],
    )
    return kfn(cnt64, adj1m)


def _lin_scale_body(src_ref, w_ref, dis_ref, o_ref):
    acc = jnp.dot(src_ref[...], w_ref[...], preferred_element_type=jnp.float32)
    o_ref[...] = (dis_ref[...] * acc)[None, :, :]


def _lin_scale(src2d, w, dis_col, nb):
    n, bf = src2d.shape
    f = w.shape[0]
    h = w.shape[1]
    b = bf // f
    return pl.pallas_call(
        _lin_scale_body,
        grid=(b, n // nb),
        in_specs=[
            pl.BlockSpec((nb, f), lambda bi, ni: (ni, bi)),
            pl.BlockSpec((f, h), lambda bi, ni: (0, 0)),
            pl.BlockSpec((nb, 1), lambda bi, ni: (ni, 0)),
        ],
        out_specs=pl.BlockSpec((1, nb, h), lambda bi, ni: (bi, ni, 0)),
        out_shape=jax.ShapeDtypeStruct((b, n, h), jnp.float32),
    )(src2d, w, dis_col)


def _spmm_body(m_hbm, eidx_hbm, evals_hbm, ecnt_hbm, out_hbm,
               ibuf, vbuf, cg, rg, gbuf, stage, zbuf, c16, acc,
               sem0, sem1, gsem0, gsem1, *, n, h, b, rpw):
    cidx = lax.axis_index("c")
    sidx = lax.axis_index("s")
    wid = sidx * NC + cidx
    row0 = wid * rpw
    shift = (n - 1).bit_length()

    for i in range(zbuf.shape[0]):
        for j in range(h // 16):
            zbuf[i, pl.ds(j * 16, 16)] = jnp.zeros((16,), jnp.float32)

    pltpu.sync_copy(ecnt_hbm.at[wid], c16)
    cvec16 = c16[...]
    sems = [sem0, sem1]
    gsems = [gsem0, gsem1]
    halfn = n // 2
    rows_per_tile = halfn // NS
    for half in range(2):
        pltpu.sync_copy(eidx_hbm.at[wid, half], ibuf)
        pltpu.sync_copy(evals_hbm.at[wid, half], vbuf)
        cnt = cvec16[half]
        ngr = cnt // GK

        def _unpack(i, _):
            flat = ibuf[pl.ds(i * 16, 16)]
            cg[i, :] = (flat & (n - 1)) - half * halfn
            return 0

        lax.fori_loop(0, ngr, _unpack, 0)

        for bi in range(b):
            def _rows(i, _):
                flat = ibuf[pl.ds(i * 16, 16)]
                rg[i, :] = (bi * n + row0) + lax.shift_right_logical(
                    flat, shift)
                return 0

            lax.fori_loop(0, ngr, _rows, 0)
            for t in range(rows_per_tile // zbuf.shape[0]):
                pltpu.sync_copy(
                    zbuf,
                    acc.at[pl.ds(sidx * rows_per_tile + t * zbuf.shape[0],
                                 zbuf.shape[0])])
            plsc.subcore_barrier()

            for s in range(2):
                @pl.when(s < ngr)
                def _():
                    pltpu.async_copy(m_hbm.at[rg.at[s]], gbuf.at[s],
                                     gsems[s])

            def _g2(g2, _):
                for s in range(2):
                    g = g2 * 2 + s
                    pltpu.make_async_copy(
                        m_hbm.at[rg.at[0]], gbuf.at[s], gsems[s]).wait()

                    @pl.when(g2 > 0)
                    def _():
                        pltpu.make_async_copy(
                            stage.at[s], acc.at[cg.at[0]], sems[s]).wait()

                    wvec = vbuf[pl.ds(g * GK, GK)]
                    for e in range(GK):
                        w = wvec[e]
                        for j in range(h // 16):
                            stage[s, e, pl.ds(j * 16, 16)] = (
                                gbuf[s, e, pl.ds(j * 16, 16)] * w)

                    @pl.when(g + 2 < ngr)
                    def _():
                        pltpu.async_copy(m_hbm.at[rg.at[g + 2]], gbuf.at[s],
                                         gsems[s])
                    pltpu.async_copy(stage.at[s], acc.at[cg.at[g]], sems[s],
                                     add=True)
                return 0

            lax.fori_loop(0, ngr // 2, _g2, 0)
            for s in range(2):
                @pl.when(ngr > s)
                def _():
                    pltpu.make_async_copy(
                        stage.at[s], acc.at[cg.at[0]], sems[s]).wait()
            plsc.subcore_barrier()
            off = half * halfn + sidx * rows_per_tile
            pltpu.sync_copy(
                acc.at[pl.ds(sidx * rows_per_tile, rows_per_tile)],
                out_hbm.at[bi, cidx, pl.ds(off, rows_per_tile)])
            plsc.subcore_barrier()


def _spmm_sc(m3, eidx, evals, ecnt):
    b, n, h = m3.shape
    rpw = n // NW
    mesh = plsc.VectorSubcoreMesh(core_axis_name="c", subcore_axis_name="s",
                                  num_cores=NC, num_subcores=NS)
    kfn = pl.kernel(
        functools.partial(_spmm_body, n=n, h=h, b=b, rpw=rpw),
        out_type=jax.ShapeDtypeStruct((b, NC, n, h), jnp.float32),
        mesh=mesh,
        compiler_params=pltpu.CompilerParams(needs_layout_passes=False),
        scratch_types=[
            pltpu.VMEM((CAP,), jnp.int32),
            pltpu.VMEM((CAP,), jnp.float32),
            pltpu.VMEM((CAP // 16, 16), jnp.int32),
            pltpu.VMEM((CAP // 16, 16), jnp.int32),
            pltpu.VMEM((2, GK, h), jnp.float32),
            pltpu.VMEM((2, GK, h), jnp.float32),
            pltpu.VMEM((8, h), jnp.float32),
            pltpu.VMEM((16,), jnp.int32),
            pltpu.VMEM_SHARED((n // 2, h), jnp.float32),
            pltpu.SemaphoreType.DMA,
            pltpu.SemaphoreType.DMA,
            pltpu.SemaphoreType.DMA,
            pltpu.SemaphoreType.DMA,
        ],
    )
    return kfn(m3.reshape(b * n, h), eidx, evals, ecnt)


def _merge_body(p_ref, m_ref, dis_ref, bias_ref, h_ref, st_ref, *, nblk):
    ni = pl.program_id(1)
    acc = p_ref[0, 0] + p_ref[0, 1] + m_ref[0]
    hv = jnp.maximum(dis_ref[...] * acc + bias_ref[...], 0.0)
    h_ref[...] = hv
    s1 = jnp.sum(hv, axis=0, keepdims=True)
    s2 = jnp.sum(hv * hv, axis=0, keepdims=True)
    st = jnp.concatenate([s1, s2], axis=0)

    @pl.when(ni == 0)
    def _():
        st_ref[...] = st

    @pl.when(ni > 0)
    def _():
        st_ref[...] += st


def _merge(p, m3, dis_col, bias, nb):
    b, _, n, h = p.shape
    nblk = n // nb
    return pl.pallas_call(
        functools.partial(_merge_body, nblk=nblk),
        grid=(b, nblk),
        in_specs=[
            pl.BlockSpec((1, NC, nb, h), lambda bi, ni: (bi, 0, ni, 0)),
            pl.BlockSpec((1, nb, h), lambda bi, ni: (bi, ni, 0)),
            pl.BlockSpec((nb, 1), lambda bi, ni: (ni, 0)),
            pl.BlockSpec((1, h), lambda bi, ni: (0, 0)),
        ],
        out_specs=[
            pl.BlockSpec((nb, h), lambda bi, ni: (ni, bi)),
            pl.BlockSpec((2, h), lambda bi, ni: (0, bi)),
        ],
        out_shape=[
            jax.ShapeDtypeStruct((n, b * h), jnp.float32),
            jax.ShapeDtypeStruct((2, b * h), jnp.float32),
        ],
    )(p, m3, dis_col, bias)


def _bn_body(h_ref, st_ref, g_ref, b_ref, o_ref, *, count, b):
    f = g_ref.shape[1]
    st = st_ref[...]
    s1 = sum(st[0:1, i * f:(i + 1) * f] for i in range(b))
    s2 = sum(st[1:2, i * f:(i + 1) * f] for i in range(b))
    mean = s1 / count
    var = s2 / count - mean * mean
    inv = lax.rsqrt(var + EPS)
    s = g_ref[...] * inv
    t = b_ref[...] - mean * s
    s_t = jnp.concatenate([s] * b, axis=1)
    t_t = jnp.concatenate([t] * b, axis=1)
    o_ref[...] = h_ref[...] * s_t + t_t


def _bn_apply(h2d, st, g, bb, nb):
    n, bf = h2d.shape
    f = g.shape[1]
    b = bf // f
    return pl.pallas_call(
        functools.partial(_bn_body, count=float(n * b), b=b),
        grid=(n // nb,),
        in_specs=[
            pl.BlockSpec((nb, bf), lambda ni: (ni, 0)),
            pl.BlockSpec((2, bf), lambda ni: (0, 0)),
            pl.BlockSpec((1, f), lambda ni: (0, 0)),
            pl.BlockSpec((1, f), lambda ni: (0, 0)),
        ],
        out_specs=pl.BlockSpec((nb, bf), lambda ni: (ni, 0)),
        out_shape=jax.ShapeDtypeStruct((n, bf), jnp.float32),
    )(h2d, st, g, bb)


def _fc_body(x_ref, h1_ref, h2_ref, w1_ref, b1_ref, w2_ref, b2_ref, o_ref,
             *, f, hdim):
    wa = w1_ref[0:f, :]
    wb = w1_ref[f:f + hdim, :]
    wc = w1_ref[f + hdim:f + 2 * hdim, :]
    z = jnp.dot(x_ref[...], wa, preferred_element_type=jnp.float32)
    z += jnp.dot(h1_ref[...], wb, preferred_element_type=jnp.float32)
    z += jnp.dot(h2_ref[...], wc, preferred_element_type=jnp.float32)
    z = jnp.maximum(z + b1_ref[...], 0.0)
    z2 = jnp.dot(z, w2_ref[...], preferred_element_type=jnp.float32)
    z2 = jnp.maximum(z2 + b2_ref[...], 0.0)
    o_ref[...] = z2[None, :, :]


def _fc(x2d, h1, h2, w1, b1, w2, b2, nb, f):
    n, bf = x2d.shape
    hdim = (w1.shape[0] - f) // 2
    b = bf // f
    nout = w2.shape[1]
    return pl.pallas_call(
        functools.partial(_fc_body, f=f, hdim=hdim),
        grid=(b, n // nb),
        in_specs=[
            pl.BlockSpec((nb, f), lambda bi, ni: (ni, bi)),
            pl.BlockSpec((nb, hdim), lambda bi, ni: (ni, bi)),
            pl.BlockSpec((nb, hdim), lambda bi, ni: (ni, bi)),
            pl.BlockSpec((f + 2 * hdim, w1.shape[1]), lambda bi, ni: (0, 0)),
            pl.BlockSpec((1, w1.shape[1]), lambda bi, ni: (0, 0)),
            pl.BlockSpec((w1.shape[1], nout), lambda bi, ni: (0, 0)),
            pl.BlockSpec((1, nout), lambda bi, ni: (0, 0)),
        ],
        out_specs=pl.BlockSpec((1, nb, nout), lambda bi, ni: (bi, ni, 0)),
        out_shape=jax.ShapeDtypeStruct((b, n, nout), jnp.float32),
    )(x2d, h1, h2, w1, b1, w2, b2)


def kernel(adj, x, conv1_W, conv1_b, conv2_W, conv2_b, bn1_g, bn1_b,
           bn2_g, bn2_b, fc1_W, fc1_b, fc2_W, fc2_b):
    n = adj.shape[0]
    b, _, f = x.shape
    rb = min(256, n)
    nb = min(512, n)

    sel = (jnp.arange(n)[:, None] // CH
           == jnp.arange(n // CH)[None, :]).astype(jnp.bfloat16)
    dis_row, cnt64 = _prepass(adj, sel, rb)
    dis_col = dis_row.reshape(n, 1)

    adj1m = adj.reshape(n * (n // CH), CH)
    eidx, evals, ecnt = _extract(cnt64, adj1m, n)

    x2d = _xpose(x, nb)

    m1 = _lin_scale(x2d, conv1_W, dis_col, nb)
    p1 = _spmm_sc(m1, eidx, evals, ecnt)
    h1, st1 = _merge(p1, m1, dis_col, conv1_b.reshape(1, -1), nb)
    h1h = _bn_apply(h1, st1, bn1_g.reshape(1, -1), bn1_b.reshape(1, -1), nb)

    m2 = _lin_scale(h1h, conv2_W, dis_col, nb)
    p2 = _spmm_sc(m2, eidx, evals, ecnt)
    h2, st2 = _merge(p2, m2, dis_col, conv2_b.reshape(1, -1), nb)
    h2h = _bn_apply(h2, st2, bn2_g.reshape(1, -1), bn2_b.reshape(1, -1), nb)

    out = _fc(x2d, h1h, h2h, fc1_W, fc1_b.reshape(1, -1),
              fc2_W, fc2_b.reshape(1, -1), nb, f)
    return out


# speedup vs baseline: 1.1248x; 1.1248x over previous
import functools

import jax
import jax.numpy as jnp
from jax import lax
from jax.experimental import pallas as pl
import jax.experimental.pallas.tpu as pltpu
from jax.experimental.pallas import tpu_sc as plsc

EPS = 1e-5

NC = 2
NS = 16
NW = NC * NS
CH = 128
CAP = 4096
EPAD = 32
GK = 16


def _prepass_body(adj_ref, sel_ref, dis_ref, cnt_ref, *, nsteps):
    i = pl.program_id(0)
    blk = adj_ref[...]
    part = jnp.sum(blk, axis=0, keepdims=True)

    @pl.when(i == 0)
    def _():
        dis_ref[...] = part

    @pl.when(i > 0)
    def _():
        dis_ref[...] += part

    @pl.when(i == nsteps - 1)
    def _():
        dis_ref[...] = lax.rsqrt(1.0 + dis_ref[...])

    nz = (blk != 0.0).astype(jnp.bfloat16)
    cnt = jnp.dot(nz, sel_ref[...], preferred_element_type=jnp.float32)
    cnt_ref[...] = cnt.astype(jnp.int32)


def _prepass(adj, sel, rb):
    n = adj.shape[0]
    nsteps = n // rb
    nch = n // CH
    return pl.pallas_call(
        functools.partial(_prepass_body, nsteps=nsteps),
        grid=(nsteps,),
        in_specs=[
            pl.BlockSpec((rb, n), lambda i: (i, 0)),
            pl.BlockSpec((n, nch), lambda i: (0, 0)),
        ],
        out_specs=[
            pl.BlockSpec((1, n), lambda i: (0, 0)),
            pl.BlockSpec((rb, nch), lambda i: (i, 0)),
        ],
        out_shape=[
            jax.ShapeDtypeStruct((1, n), jnp.float32),
            jax.ShapeDtypeStruct((n, nch), jnp.int32),
        ],
    )(adj, sel)


def _xpose_body(x_ref, o_ref):
    o_ref[...] = x_ref[...]


def _xpose(x, nb):
    b, n, f = x.shape
    nblk = n // nb
    return pl.pallas_call(
        _xpose_body,
        grid=(b, nblk),
        in_specs=[pl.BlockSpec((nb, f), lambda bi, ni: (bi * nblk + ni, 0))],
        out_specs=pl.BlockSpec((nb, f), lambda bi, ni: (ni, bi)),
        out_shape=jax.ShapeDtypeStruct((n, b * f), jnp.float32),
    )(x.reshape(b * n, f))


def _cstore(ref, base, x, msk):
    cs = plsc.cumsum(msk.astype(jnp.int32))
    plsc.store_scatter(ref, [base + cs - 1], x, mask=msk)
    return cs[15]


def _extract_body(cnt_hbm, adj1m_hbm, eidx_hbm, evals_hbm, ecnt_hbm,
                  cntbuf, cidA, cidB, chk, ibuf, vbuf, c16, gsem0, gsem1,
                  *, n, rpw):
    cidx = lax.axis_index("c")
    sidx = lax.axis_index("s")
    wid = sidx * NC + cidx
    row0 = wid * rpw
    ncpr = n // CH
    halfn = n // 2
    iota = lax.iota(jnp.int32, 16)
    zeros_i = jnp.zeros((16,), jnp.int32)
    zeros_f = jnp.zeros((16,), jnp.float32)

    pltpu.sync_copy(cnt_hbm.at[pl.ds(row0, rpw)], cntbuf)

    def _cchunk(r, carry):
        cA, cB = carry
        for k in range(ncpr // 16):
            v = cntbuf[r, pl.ds(k * 16, 16)]
            msk = v > 0
            ids = (row0 + r) * ncpr + k * 16 + iota
            if k < ncpr // 32:
                cA = cA + _cstore(cidA, cA, ids, msk)
            else:
                cB = cB + _cstore(cidB, cB, ids, msk)
        return (cA, cB)

    cA, cB = lax.fori_loop(0, rpw, _cchunk,
                           (jnp.int32(0), jnp.int32(0)))

    gsems = [gsem0, gsem1]
    for half, (cid, cptr) in enumerate(((cidA, cA), (cidB, cB))):
        cpad = (cptr + GK - 1) // GK * GK
        _cstore(cid, cptr, zeros_i, iota < (cpad - cptr))
        ngr = cpad // GK

        @pl.when(ngr > 0)
        def _():
            pltpu.async_copy(adj1m_hbm.at[cid.at[pl.ds(0, GK)]],
                             chk.at[0], gsems[0])

        def _group(g, eptr):
            s = g % 2
            for sj in range(2):
                @pl.when(s == sj)
                def _():
                    pltpu.make_async_copy(
                        adj1m_hbm.at[cid.at[pl.ds(0, GK)]],
                        chk.at[sj], gsems[sj]).wait()

                @pl.when((s == sj) & (g + 1 < ngr))
                def _():
                    pltpu.async_copy(
                        adj1m_hbm.at[cid.at[pl.ds((g + 1) * GK, GK)]],
                        chk.at[1 - sj], gsems[1 - sj])

            cvec = cid[pl.ds(g * GK, GK)]
            for j in range(GK):
                ce = g * GK + j
                live = ce < cptr
                cj = cvec[j]
                base = cj * CH - row0 * n

                def _sub(jj, eptr):
                    x = chk[s, j, pl.ds(jj * 16, 16)]
                    msk = (x != 0.0) & live & (eptr < CAP)
                    cntv = jnp.sum(msk.astype(jnp.int32))

                    @pl.when(cntv > 0)
                    def _():
                        _cstore(vbuf, eptr, x, msk)
                        _cstore(ibuf, eptr, base + jj * 16 + iota, msk)

                    return eptr + cntv

                eptr = lax.fori_loop(0, CH // 16, _sub, eptr)
            return eptr

        eptr = lax.fori_loop(0, ngr, _group, jnp.int32(0))

        pad_id = zeros_i + half * halfn
        epad = (eptr + EPAD - 1) // EPAD * EPAD
        padn = epad - eptr
        m1 = iota < jnp.minimum(padn, 16)
        _cstore(ibuf, eptr, pad_id, m1)
        _cstore(vbuf, eptr, zeros_f, m1)
        e2 = eptr + jnp.minimum(padn, 16)
        m2 = iota < (padn - 16)
        _cstore(ibuf, e2, pad_id, m2)
        _cstore(vbuf, e2, zeros_f, m2)

        pltpu.sync_copy(ibuf.at[pl.ds(0, CAP)], eidx_hbm.at[wid, half])
        pltpu.sync_copy(vbuf.at[pl.ds(0, CAP)], evals_hbm.at[wid, half])
        lane = jnp.where(iota == half, epad, 0).astype(jnp.int32)
        if half == 0:
            c16[...] = lane
        else:
            c16[...] += lane
    pltpu.sync_copy(c16, ecnt_hbm.at[wid])


def _extract(cnt64, adj1m, n):
    rpw = n // NW
    nhchunk = rpw * (n // CH) // 2
    mesh = plsc.VectorSubcoreMesh(core_axis_name="c", subcore_axis_name="s",
                                  num_cores=NC, num_subcores=NS)
    kfn = pl.kernel(
        functools.partial(_extract_body, n=n, rpw=rpw),
        out_type=[
            jax.ShapeDtypeStruct((NW, 2, CAP), jnp.int32),
            jax.ShapeDtypeStruct((NW, 2, CAP), jnp.float32),
            jax.ShapeDtypeStruct((NW, 16), jnp.int32),
        ],
        mesh=mesh,
        compiler_params=pltpu.CompilerParams(needs_layout_passes=False),
        scratch_types=[
            pltpu.VMEM((rpw, n // CH), jnp.int32),
            pltpu.VMEM((nhchunk + 16,), jnp.int32),
            pltpu.VMEM((nhchunk + 16,), jnp.int32),
            pltpu.VMEM((2, GK, CH), jnp.float32),
            pltpu.VMEM((CAP + 16,), jnp.int32),
            pltpu.VMEM((CAP + 16,), jnp.float32),
            pltpu.VMEM((16,), jnp.int32),
            pltpu.SemaphoreType.DMA,
            pltpu.SemaphoreType.DMA,
        ],
    )
    return kfn(cnt64, adj1m)


def _lin_scale_body(src_ref, w_ref, dis_ref, o_ref):
    acc = jnp.dot(src_ref[...], w_ref[...], preferred_element_type=jnp.float32)
    o_ref[...] = (dis_ref[...] * acc)[None, :, :]


def _lin_scale(src2d, w, dis_col, nb):
    n, bf = src2d.shape
    f = w.shape[0]
    h = w.shape[1]
    b = bf // f
    return pl.pallas_call(
        _lin_scale_body,
        grid=(b, n // nb),
        in_specs=[
            pl.BlockSpec((nb, f), lambda bi, ni: (ni, bi)),
            pl.BlockSpec((f, h), lambda bi, ni: (0, 0)),
            pl.BlockSpec((nb, 1), lambda bi, ni: (ni, 0)),
        ],
        out_specs=pl.BlockSpec((1, nb, h), lambda bi, ni: (bi, ni, 0)),
        out_shape=jax.ShapeDtypeStruct((b, n, h), jnp.float32),
    )(src2d, w, dis_col)


def _spmm_body(m_hbm, eidx_hbm, evals_hbm, ecnt_hbm, out_hbm,
               mslab, ibuf, vbuf, cg, stage, zbuf, c16, acc,
               sem0, sem1, *, n, h, b, rpw):
    cidx = lax.axis_index("c")
    sidx = lax.axis_index("s")
    wid = sidx * NC + cidx
    row0 = wid * rpw
    shift = (n - 1).bit_length()

    for i in range(zbuf.shape[0]):
        for j in range(h // 16):
            zbuf[i, pl.ds(j * 16, 16)] = jnp.zeros((16,), jnp.float32)

    pltpu.sync_copy(ecnt_hbm.at[wid], c16)
    cvec16 = c16[...]
    sems = [sem0, sem1]
    halfn = n // 2
    rows_per_tile = halfn // NS
    for half in range(2):
        pltpu.sync_copy(eidx_hbm.at[wid, half], ibuf)
        pltpu.sync_copy(evals_hbm.at[wid, half], vbuf)
        cnt = cvec16[half]
        ngr = cnt // GK

        def _unpack(i, _):
            flat = ibuf[pl.ds(i * 16, 16)]
            cg[i, :] = (flat & (n - 1)) - half * halfn
            return 0

        lax.fori_loop(0, ngr, _unpack, 0)

        def _batch(bi, _):
            pltpu.sync_copy(m_hbm.at[pl.ds(bi * n + row0, rpw)], mslab)
            for t in range(rows_per_tile // zbuf.shape[0]):
                pltpu.sync_copy(
                    zbuf,
                    acc.at[pl.ds(sidx * rows_per_tile + t * zbuf.shape[0],
                                 zbuf.shape[0])])
            plsc.subcore_barrier()

            def _g2(g2, _):
                for s in range(2):
                    g = g2 * 2 + s

                    @pl.when(g2 > 0)
                    def _():
                        pltpu.make_async_copy(
                            stage.at[s], acc.at[cg.at[0]], sems[s]).wait()

                    rvec = lax.shift_right_logical(
                        ibuf[pl.ds(g * GK, GK)], shift)
                    wvec = vbuf[pl.ds(g * GK, GK)]
                    for e in range(GK):
                        r = rvec[e]
                        w = wvec[e]
                        for j in range(h // 16):
                            stage[s, e, pl.ds(j * 16, 16)] = (
                                mslab[r, pl.ds(j * 16, 16)] * w)
                    pltpu.async_copy(stage.at[s], acc.at[cg.at[g]], sems[s],
                                     add=True)
                return 0

            lax.fori_loop(0, ngr // 2, _g2, 0)
            for s in range(2):
                @pl.when(ngr > s)
                def _():
                    pltpu.make_async_copy(
                        stage.at[s], acc.at[cg.at[0]], sems[s]).wait()
            plsc.subcore_barrier()
            off = half * halfn + sidx * rows_per_tile
            pltpu.sync_copy(
                acc.at[pl.ds(sidx * rows_per_tile, rows_per_tile)],
                out_hbm.at[bi, cidx, pl.ds(off, rows_per_tile)])
            plsc.subcore_barrier()
            return 0

        lax.fori_loop(0, b, _batch, 0)


def _spmm_sc(m3, eidx, evals, ecnt):
    b, n, h = m3.shape
    rpw = n // NW
    mesh = plsc.VectorSubcoreMesh(core_axis_name="c", subcore_axis_name="s",
                                  num_cores=NC, num_subcores=NS)
    kfn = pl.kernel(
        functools.partial(_spmm_body, n=n, h=h, b=b, rpw=rpw),
        out_type=jax.ShapeDtypeStruct((b, NC, n, h), jnp.float32),
        mesh=mesh,
        compiler_params=pltpu.CompilerParams(needs_layout_passes=False),
        scratch_types=[
            pltpu.VMEM((rpw, h), jnp.float32),
            pltpu.VMEM((CAP,), jnp.int32),
            pltpu.VMEM((CAP,), jnp.float32),
            pltpu.VMEM((CAP // 16, 16), jnp.int32),
            pltpu.VMEM((2, GK, h), jnp.float32),
            pltpu.VMEM((8, h), jnp.float32),
            pltpu.VMEM((16,), jnp.int32),
            pltpu.VMEM_SHARED((n // 2, h), jnp.float32),
            pltpu.SemaphoreType.DMA,
            pltpu.SemaphoreType.DMA,
        ],
    )
    return kfn(m3.reshape(b * n, h), eidx, evals, ecnt)


def _merge_body(p_ref, m_ref, dis_ref, bias_ref, h_ref, st_ref, *, nblk):
    ni = pl.program_id(1)
    acc = p_ref[0, 0] + p_ref[0, 1] + m_ref[0]
    hv = jnp.maximum(dis_ref[...] * acc + bias_ref[...], 0.0)
    h_ref[...] = hv
    s1 = jnp.sum(hv, axis=0, keepdims=True)
    s2 = jnp.sum(hv * hv, axis=0, keepdims=True)
    st = jnp.concatenate([s1, s2], axis=0)

    @pl.when(ni == 0)
    def _():
        st_ref[...] = st

    @pl.when(ni > 0)
    def _():
        st_ref[...] += st


def _merge(p, m3, dis_col, bias, nb):
    b, _, n, h = p.shape
    nblk = n // nb
    return pl.pallas_call(
        functools.partial(_merge_body, nblk=nblk),
        grid=(b, nblk),
        in_specs=[
            pl.BlockSpec((1, NC, nb, h), lambda bi, ni: (bi, 0, ni, 0)),
            pl.BlockSpec((1, nb, h), lambda bi, ni: (bi, ni, 0)),
            pl.BlockSpec((nb, 1), lambda bi, ni: (ni, 0)),
            pl.BlockSpec((1, h), lambda bi, ni: (0, 0)),
        ],
        out_specs=[
            pl.BlockSpec((nb, h), lambda bi, ni: (ni, bi)),
            pl.BlockSpec((2, h), lambda bi, ni: (0, bi)),
        ],
        out_shape=[
            jax.ShapeDtypeStruct((n, b * h), jnp.float32),
            jax.ShapeDtypeStruct((2, b * h), jnp.float32),
        ],
    )(p, m3, dis_col, bias)


def _bn_body(h_ref, st_ref, g_ref, b_ref, o_ref, *, count, b):
    f = g_ref.shape[1]
    st = st_ref[...]
    s1 = sum(st[0:1, i * f:(i + 1) * f] for i in range(b))
    s2 = sum(st[1:2, i * f:(i + 1) * f] for i in range(b))
    mean = s1 / count
    var = s2 / count - mean * mean
    inv = lax.rsqrt(var + EPS)
    s = g_ref[...] * inv
    t = b_ref[...] - mean * s
    s_t = jnp.concatenate([s] * b, axis=1)
    t_t = jnp.concatenate([t] * b, axis=1)
    o_ref[...] = h_ref[...] * s_t + t_t


def _bn_apply(h2d, st, g, bb, nb):
    n, bf = h2d.shape
    f = g.shape[1]
    b = bf // f
    return pl.pallas_call(
        functools.partial(_bn_body, count=float(n * b), b=b),
        grid=(n // nb,),
        in_specs=[
            pl.BlockSpec((nb, bf), lambda ni: (ni, 0)),
            pl.BlockSpec((2, bf), lambda ni: (0, 0)),
            pl.BlockSpec((1, f), lambda ni: (0, 0)),
            pl.BlockSpec((1, f), lambda ni: (0, 0)),
        ],
        out_specs=pl.BlockSpec((nb, bf), lambda ni: (ni, 0)),
        out_shape=jax.ShapeDtypeStruct((n, bf), jnp.float32),
    )(h2d, st, g, bb)


def _fc_body(x_ref, h1_ref, h2_ref, w1_ref, b1_ref, w2_ref, b2_ref, o_ref,
             *, f, hdim):
    wa = w1_ref[0:f, :]
    wb = w1_ref[f:f + hdim, :]
    wc = w1_ref[f + hdim:f + 2 * hdim, :]
    z = jnp.dot(x_ref[...], wa, preferred_element_type=jnp.float32)
    z += jnp.dot(h1_ref[...], wb, preferred_element_type=jnp.float32)
    z += jnp.dot(h2_ref[...], wc, preferred_element_type=jnp.float32)
    z = jnp.maximum(z + b1_ref[...], 0.0)
    z2 = jnp.dot(z, w2_ref[...], preferred_element_type=jnp.float32)
    z2 = jnp.maximum(z2 + b2_ref[...], 0.0)
    o_ref[...] = z2[None, :, :]


def _fc(x2d, h1, h2, w1, b1, w2, b2, nb, f):
    n, bf = x2d.shape
    hdim = (w1.shape[0] - f) // 2
    b = bf // f
    nout = w2.shape[1]
    return pl.pallas_call(
        functools.partial(_fc_body, f=f, hdim=hdim),
        grid=(b, n // nb),
        in_specs=[
            pl.BlockSpec((nb, f), lambda bi, ni: (ni, bi)),
            pl.BlockSpec((nb, hdim), lambda bi, ni: (ni, bi)),
            pl.BlockSpec((nb, hdim), lambda bi, ni: (ni, bi)),
            pl.BlockSpec((f + 2 * hdim, w1.shape[1]), lambda bi, ni: (0, 0)),
            pl.BlockSpec((1, w1.shape[1]), lambda bi, ni: (0, 0)),
            pl.BlockSpec((w1.shape[1], nout), lambda bi, ni: (0, 0)),
            pl.BlockSpec((1, nout), lambda bi, ni: (0, 0)),
        ],
        out_specs=pl.BlockSpec((1, nb, nout), lambda bi, ni: (bi, ni, 0)),
        out_shape=jax.ShapeDtypeStruct((b, n, nout), jnp.float32),
    )(x2d, h1, h2, w1, b1, w2, b2)


def kernel(adj, x, conv1_W, conv1_b, conv2_W, conv2_b, bn1_g, bn1_b,
           bn2_g, bn2_b, fc1_W, fc1_b, fc2_W, fc2_b):
    n = adj.shape[0]
    b, _, f = x.shape
    rb = min(256, n)
    nb = min(512, n)

    sel = (jnp.arange(n)[:, None] // CH
           == jnp.arange(n // CH)[None, :]).astype(jnp.bfloat16)
    dis_row, cnt64 = _prepass(adj, sel, rb)
    dis_col = dis_row.reshape(n, 1)

    adj1m = adj.reshape(n * (n // CH), CH)
    eidx, evals, ecnt = _extract(cnt64, adj1m, n)

    x2d = _xpose(x, nb)

    m1 = _lin_scale(x2d, conv1_W, dis_col, nb)
    p1 = _spmm_sc(m1, eidx, evals, ecnt)
    h1, st1 = _merge(p1, m1, dis_col, conv1_b.reshape(1, -1), nb)
    h1h = _bn_apply(h1, st1, bn1_g.reshape(1, -1), bn1_b.reshape(1, -1), nb)

    m2 = _lin_scale(h1h, conv2_W, dis_col, nb)
    p2 = _spmm_sc(m2, eidx, evals, ecnt)
    h2, st2 = _merge(p2, m2, dis_col, conv2_b.reshape(1, -1), nb)
    h2h = _bn_apply(h2, st2, bn2_g.reshape(1, -1), bn2_b.reshape(1, -1), nb)

    out = _fc(x2d, h1h, h2h, fc1_W, fc1_b.reshape(1, -1),
              fc2_W, fc2_b.reshape(1, -1), nb, f)
    return out
